# Initial kernel scaffold; baseline (speedup 1.0000x reference)
import jax
import jax.numpy as jnp
from jax.experimental import pallas as pl


def kernel(x, edge_index, edge_attr, batch, atom_tables, W1, b1, g1, be1, W2, b2, eps, g2, be2, Wf, bf):
    raise NotImplementedError("write your pallas kernel here")


# speedup vs baseline: 3.0691x; 3.0691x over previous
import functools

import jax
import jax.numpy as jnp
from jax import lax
from jax.experimental import pallas as pl
from jax.experimental.pallas import tpu as pltpu
from jax.experimental.pallas import tpu_sc as plsc

N = 10000
E = 160000
H = 256
G = 256
OUT = 128
F = 9
VOCAB = 64

NSUB = 16
NCORE = 2
EP = E // NSUB
K = 80
NBATCH = EP // K
NPAD = 10240
RP = NPAD // NSUB

BN = 1000
NB = N // BN

VPAD = 640


def _sc_scatter_body(h2_hbm, src2_hbm, dst_hbm, zeros_hbm, out_hbm,
                     sidx, didx, rows, agg_sh, sem):
    c = lax.axis_index("c")
    s = lax.axis_index("s")
    pltpu.sync_copy(zeros_hbm.at[pl.ds(s * RP, RP)], agg_sh.at[pl.ds(s * RP, RP)])
    plsc.subcore_barrier()

    def body(j, _):
        base = pl.multiple_of(s * EP + j * K, 8)
        pltpu.sync_copy(src2_hbm.at[pl.ds(c * E + base, K)], sidx)
        pltpu.sync_copy(dst_hbm.at[pl.ds(base, K)], didx)
        pltpu.async_copy(h2_hbm.at[sidx], rows, sem).wait()
        pltpu.sync_copy(rows, agg_sh.at[didx], add=True)
        return ()

    lax.fori_loop(0, NBATCH, body, (), unroll=False)
    plsc.subcore_barrier()
    pltpu.sync_copy(agg_sh.at[pl.ds(s * RP, RP)],
                    out_hbm.at[pl.ds(c * NPAD + s * RP, RP)])


@functools.cache
def _get_sc_scatter():
    return pl.kernel(
        _sc_scatter_body,
        out_type=jax.ShapeDtypeStruct((2 * NPAD, 128), jnp.float32),
        mesh=plsc.VectorSubcoreMesh(core_axis_name="c", subcore_axis_name="s"),
        scratch_types=[
            pltpu.VMEM((K,), jnp.int32),
            pltpu.VMEM((K,), jnp.int32),
            pltpu.VMEM((K, 128), jnp.float32),
            pltpu.VMEM_SHARED((NPAD, 128), jnp.float32),
            pltpu.SemaphoreType.DMA,
        ],
    )


def _sc_scatter(h2, src2, dst, zeros):
    return _get_sc_scatter()(h2, src2, dst, zeros)


def _encoder_body(xfT_ref, tab_ref, out_ref):
    ohT = jnp.zeros((VPAD, BN), jnp.float32)
    iota = lax.broadcasted_iota(jnp.int32, (VPAD, BN), 0)
    for i in range(F):
        row = xfT_ref[0, i, :]
        ohT = ohT + (iota == row[None, :]).astype(jnp.float32)
    out_ref[...] = lax.dot_general(
        ohT, tab_ref[...], (((0,), (0,)), ((), ())),
        preferred_element_type=jnp.float32)


def _encoder(xfT, tab):
    return pl.pallas_call(
        _encoder_body,
        grid=(NB,),
        in_specs=[
            pl.BlockSpec((1, 16, BN), lambda i: (i, 0, 0)),
            pl.BlockSpec((VPAD, H), lambda i: (0, 0)),
        ],
        out_specs=pl.BlockSpec((BN, H), lambda i: (i, 0)),
        out_shape=jax.ShapeDtypeStruct((N, H), jnp.float32),
        compiler_params=pltpu.CompilerParams(
            dimension_semantics=("arbitrary",)),
    )(xfT, tab)


def _layer_body(h_ref, agg_ref, W1_ref, b1_ref, g1_ref, be1_ref,
                W2_ref, b2_ref, g2_ref, be2_ref, ep_ref, xs_ref,
                hnew_ref, xso_ref, z1_scr, z2_scr, st1, st2):
    p = pl.program_id(0)
    i = pl.program_id(1)

    @pl.when(p == 0)
    def _():
        epsp1 = ep_ref[0, 0]
        hblk = h_ref[...]
        z_lo = epsp1 * hblk[:, :128] + agg_ref[0]
        z_hi = epsp1 * hblk[:, 128:] + agg_ref[1]
        z1 = (lax.dot_general(z_lo, W1_ref[:128, :], (((1,), (0,)), ((), ())),
                              preferred_element_type=jnp.float32)
              + lax.dot_general(z_hi, W1_ref[128:, :], (((1,), (0,)), ((), ())),
                                preferred_element_type=jnp.float32)
              + b1_ref[...])
        z1_scr[pl.ds(i * BN, BN), :] = z1

        @pl.when(i == 0)
        def _():
            st1[...] = jnp.zeros_like(st1)

        st1[0:1, :] += jnp.sum(z1, axis=0, keepdims=True)
        st1[1:2, :] += jnp.sum(z1 * z1, axis=0, keepdims=True)

    @pl.when(p == 1)
    def _():
        mean = st1[0:1, :] * (1.0 / N)
        var = st1[1:2, :] * (1.0 / N) - mean * mean
        inv = lax.rsqrt(var + 1e-5)
        z1 = z1_scr[pl.ds(i * BN, BN), :]
        z1n = jnp.maximum((z1 - mean) * inv * g1_ref[...] + be1_ref[...], 0.0)
        z2 = lax.dot_general(z1n, W2_ref[...], (((1,), (0,)), ((), ())),
                             preferred_element_type=jnp.float32) + b2_ref[...]
        z2_scr[pl.ds(i * BN, BN), :] = z2

        @pl.when(i == 0)
        def _():
            st2[...] = jnp.zeros_like(st2)

        st2[0:1, :] += jnp.sum(z2, axis=0, keepdims=True)
        st2[1:2, :] += jnp.sum(z2 * z2, axis=0, keepdims=True)

    @pl.when(p == 2)
    def _():
        mean = st2[0:1, :] * (1.0 / N)
        var = st2[1:2, :] * (1.0 / N) - mean * mean
        inv = lax.rsqrt(var + 1e-5)
        z2 = z2_scr[pl.ds(i * BN, BN), :]
        hnew = jnp.maximum((z2 - mean) * inv * g2_ref[...] + be2_ref[...], 0.0) \
            + h_ref[...]
        hnew_ref[...] = hnew
        xso_ref[...] = xs_ref[...] + hnew


def _layer(h, agg, W1i, b1i, g1i, be1i, W2i, b2i, g2i, be2i, epsp1, xs):
    hmap = lambda p, i: (jnp.where(p == 1, 0, i), 0)
    return pl.pallas_call(
        _layer_body,
        grid=(3, NB),
        in_specs=[
            pl.BlockSpec((BN, H), hmap),
            pl.BlockSpec((2, BN, 128), lambda p, i: (0, jnp.where(p == 0, i, 0), 0)),
            pl.BlockSpec((H, 2 * H), lambda p, i: (0, 0)),
            pl.BlockSpec((1, 2 * H), lambda p, i: (0, 0)),
            pl.BlockSpec((1, 2 * H), lambda p, i: (0, 0)),
            pl.BlockSpec((1, 2 * H), lambda p, i: (0, 0)),
            pl.BlockSpec((2 * H, H), lambda p, i: (0, 0)),
            pl.BlockSpec((1, H), lambda p, i: (0, 0)),
            pl.BlockSpec((1, H), lambda p, i: (0, 0)),
            pl.BlockSpec((1, H), lambda p, i: (0, 0)),
            pl.BlockSpec((1, 1), lambda p, i: (0, 0)),
            pl.BlockSpec((BN, H), lambda p, i: (jnp.where(p == 2, i, 0), 0)),
        ],
        out_specs=[
            pl.BlockSpec((BN, H), lambda p, i: (i, 0)),
            pl.BlockSpec((BN, H), lambda p, i: (i, 0)),
        ],
        out_shape=[
            jax.ShapeDtypeStruct((N, H), jnp.float32),
            jax.ShapeDtypeStruct((N, H), jnp.float32),
        ],
        scratch_shapes=[
            pltpu.VMEM((N, 2 * H), jnp.float32),
            pltpu.VMEM((N, H), jnp.float32),
            pltpu.VMEM((8, 2 * H), jnp.float32),
            pltpu.VMEM((8, H), jnp.float32),
        ],
        compiler_params=pltpu.CompilerParams(
            dimension_semantics=("arbitrary", "arbitrary")),
    )(h, agg, W1i, b1i, g1i, be1i, W2i, b2i, g2i, be2i, epsp1, xs)


def _pool_body(xs_ref, bat_ref, Wf_ref, bf_ref, out_ref, hg_scr):
    i = pl.program_id(0)

    @pl.when(i == 0)
    def _():
        hg_scr[...] = jnp.zeros_like(hg_scr)

    ohB = (lax.broadcasted_iota(jnp.int32, (G, BN), 0)
           == bat_ref[0, 0, :][None, :]).astype(jnp.float32)
    hg_scr[...] += lax.dot_general(ohB, xs_ref[...], (((1,), (0,)), ((), ())),
                                   preferred_element_type=jnp.float32)

    @pl.when(i == NB - 1)
    def _():
        out_ref[...] = lax.dot_general(
            hg_scr[...], Wf_ref[...], (((1,), (0,)), ((), ())),
            preferred_element_type=jnp.float32) + bf_ref[...]


def _pool(xs, bat2, Wf, bf2):
    return pl.pallas_call(
        _pool_body,
        grid=(NB,),
        in_specs=[
            pl.BlockSpec((BN, H), lambda i: (i, 0)),
            pl.BlockSpec((1, 1, BN), lambda i: (i, 0, 0)),
            pl.BlockSpec((H, OUT), lambda i: (0, 0)),
            pl.BlockSpec((1, OUT), lambda i: (0, 0)),
        ],
        out_specs=pl.BlockSpec((G, OUT), lambda i: (0, 0)),
        out_shape=jax.ShapeDtypeStruct((G, OUT), jnp.float32),
        scratch_shapes=[pltpu.VMEM((G, H), jnp.float32)],
        compiler_params=pltpu.CompilerParams(
            dimension_semantics=("arbitrary",)),
    )(xs, bat2, Wf, bf2)


def kernel(x, edge_index, edge_attr, batch, atom_tables, W1, b1, g1, be1,
           W2, b2, eps, g2, be2, Wf, bf):
    del edge_attr
    xfT = (x.astype(jnp.int32).T
           + (jnp.arange(F, dtype=jnp.int32) * VOCAB)[:, None])
    xfT = jnp.concatenate(
        [xfT, jnp.full((16 - F, N), 600, jnp.int32)], axis=0)
    xfT = xfT.reshape(16, NB, BN).swapaxes(0, 1)
    tab = jnp.concatenate(
        [atom_tables.reshape(F * VOCAB, H),
         jnp.zeros((VPAD - F * VOCAB, H), jnp.float32)], axis=0)
    src = edge_index[0].astype(jnp.int32)
    dst = edge_index[1].astype(jnp.int32)
    src2 = jnp.concatenate([2 * src, 2 * src + 1])
    zeros = jnp.zeros((NPAD, 128), jnp.float32)
    bat2 = batch.astype(jnp.int32).reshape(NB, 1, BN)
    bf2 = bf.reshape(1, OUT)

    h = _encoder(xfT, tab)
    xs = h

    for i in range(W1.shape[0]):
        agg = _sc_scatter(h.reshape(2 * N, 128), src2, dst, zeros)
        agg = agg.reshape(2, NPAD, 128)
        h, xs = _layer(h, agg, W1[i], b1[i].reshape(1, 2 * H),
                       g1[i].reshape(1, 2 * H), be1[i].reshape(1, 2 * H),
                       W2[i], b2[i].reshape(1, H), g2[i].reshape(1, H),
                       be2[i].reshape(1, H),
                       (1.0 + eps[i]).reshape(1, 1), xs)

    return _pool(xs, bat2, Wf, bf2)


# speedup vs baseline: 3.6366x; 1.1849x over previous
import functools

import jax
import jax.numpy as jnp
from jax import lax
from jax.experimental import pallas as pl
from jax.experimental.pallas import tpu as pltpu
from jax.experimental.pallas import tpu_sc as plsc

N = 10000
E = 160000
H = 256
G = 256
OUT = 128
F = 9
VOCAB = 64

NSUB = 16
NCORE = 2
EP = E // NSUB
K = 128
EPP = 10240
NBATCH = EPP // K
NPH = 2
RPH = NBATCH // NPH
NBT2 = RPH // 2
NPAD = 10240
RP = NPAD // NSUB

BN = 1000
NB = N // BN

VPAD = 640


def _sc_scatter_body(h2_hbm, src2_hbm, dst_hbm, zeros_hbm, out_hbm,
                     srcv, dstv, rA, rB, agg_sh, semA, semB):
    c = lax.axis_index("c")
    s = lax.axis_index("s")
    pltpu.sync_copy(zeros_hbm.at[pl.ds(s * RP, RP)], agg_sh.at[pl.ds(s * RP, RP)])

    def stage(p):
        pltpu.sync_copy(
            src2_hbm.at[pl.ds((c * NSUB + s) * NBATCH + p * RPH, RPH)], srcv)
        pltpu.sync_copy(dst_hbm.at[pl.ds(s * NBATCH + p * RPH, RPH)], dstv)
        pltpu.async_copy(h2_hbm.at[srcv.at[0]], rA, semA)

    stage(0)
    plsc.subcore_barrier()

    def body(t, _):
        j0 = t * 2
        pltpu.async_copy(h2_hbm.at[srcv.at[j0 + 1]], rB, semB)
        pltpu.make_async_copy(h2_hbm.at[srcv.at[j0]], rA, semA).wait()
        pltpu.sync_copy(rA, agg_sh.at[dstv.at[j0]], add=True)

        @pl.when(t < NBT2 - 1)
        def _():
            pltpu.async_copy(h2_hbm.at[srcv.at[j0 + 2]], rA, semA)

        pltpu.make_async_copy(h2_hbm.at[srcv.at[j0 + 1]], rB, semB).wait()
        pltpu.sync_copy(rB, agg_sh.at[dstv.at[j0 + 1]], add=True)
        return ()

    for p in range(NPH):
        lax.fori_loop(0, NBT2, body, (), unroll=False)
        if p < NPH - 1:
            stage(p + 1)

    plsc.subcore_barrier()
    pltpu.sync_copy(agg_sh.at[pl.ds(s * RP, RP)],
                    out_hbm.at[pl.ds(c * NPAD + s * RP, RP)])


@functools.cache
def _get_sc_scatter():
    return pl.kernel(
        _sc_scatter_body,
        out_type=jax.ShapeDtypeStruct((2 * NPAD, 128), jnp.float32),
        mesh=plsc.VectorSubcoreMesh(core_axis_name="c", subcore_axis_name="s"),
        scratch_types=[
            pltpu.VMEM((RPH, K), jnp.int32),
            pltpu.VMEM((RPH, K), jnp.int32),
            pltpu.VMEM((K, 128), jnp.float32),
            pltpu.VMEM((K, 128), jnp.float32),
            pltpu.VMEM_SHARED((NPAD, 128), jnp.float32),
            pltpu.SemaphoreType.DMA,
            pltpu.SemaphoreType.DMA,
        ],
    )


def _sc_scatter(h2, src2, dst, zeros):
    return _get_sc_scatter()(h2, src2, dst, zeros)


def _encoder_body(xfT_ref, tab_ref, out_ref):
    ohT = jnp.zeros((VPAD, BN), jnp.float32)
    iota = lax.broadcasted_iota(jnp.int32, (VPAD, BN), 0)
    for i in range(F):
        row = xfT_ref[0, i, :]
        ohT = ohT + (iota == row[None, :]).astype(jnp.float32)
    out_ref[...] = lax.dot_general(
        ohT, tab_ref[...], (((0,), (0,)), ((), ())),
        preferred_element_type=jnp.float32)


def _encoder(xfT, tab):
    return pl.pallas_call(
        _encoder_body,
        grid=(NB,),
        in_specs=[
            pl.BlockSpec((1, 16, BN), lambda i: (i, 0, 0)),
            pl.BlockSpec((VPAD, H), lambda i: (0, 0)),
        ],
        out_specs=pl.BlockSpec((BN, H), lambda i: (i, 0)),
        out_shape=jax.ShapeDtypeStruct((N, H), jnp.float32),
        compiler_params=pltpu.CompilerParams(
            dimension_semantics=("arbitrary",)),
    )(xfT, tab)


def _layer_body(h_ref, agg_ref, W1_ref, b1_ref, g1_ref, be1_ref,
                W2_ref, b2_ref, g2_ref, be2_ref, ep_ref, xs_ref,
                hnew_ref, xso_ref, z1_scr, z2_scr, st1, st2):
    p = pl.program_id(0)
    i = pl.program_id(1)

    @pl.when(p == 0)
    def _():
        epsp1 = ep_ref[0, 0]
        hblk = h_ref[...]
        z_lo = epsp1 * hblk[:, :128] + agg_ref[0]
        z_hi = epsp1 * hblk[:, 128:] + agg_ref[1]
        z1 = (lax.dot_general(z_lo, W1_ref[:128, :], (((1,), (0,)), ((), ())),
                              preferred_element_type=jnp.float32)
              + lax.dot_general(z_hi, W1_ref[128:, :], (((1,), (0,)), ((), ())),
                                preferred_element_type=jnp.float32)
              + b1_ref[...])
        z1_scr[pl.ds(i * BN, BN), :] = z1

        @pl.when(i == 0)
        def _():
            st1[...] = jnp.zeros_like(st1)

        st1[0:1, :] += jnp.sum(z1, axis=0, keepdims=True)
        st1[1:2, :] += jnp.sum(z1 * z1, axis=0, keepdims=True)

    @pl.when(p == 1)
    def _():
        mean = st1[0:1, :] * (1.0 / N)
        var = st1[1:2, :] * (1.0 / N) - mean * mean
        inv = lax.rsqrt(var + 1e-5)
        z1 = z1_scr[pl.ds(i * BN, BN), :]
        z1n = jnp.maximum((z1 - mean) * inv * g1_ref[...] + be1_ref[...], 0.0)
        z2 = lax.dot_general(z1n, W2_ref[...], (((1,), (0,)), ((), ())),
                             preferred_element_type=jnp.float32) + b2_ref[...]
        z2_scr[pl.ds(i * BN, BN), :] = z2

        @pl.when(i == 0)
        def _():
            st2[...] = jnp.zeros_like(st2)

        st2[0:1, :] += jnp.sum(z2, axis=0, keepdims=True)
        st2[1:2, :] += jnp.sum(z2 * z2, axis=0, keepdims=True)

    @pl.when(p == 2)
    def _():
        mean = st2[0:1, :] * (1.0 / N)
        var = st2[1:2, :] * (1.0 / N) - mean * mean
        inv = lax.rsqrt(var + 1e-5)
        z2 = z2_scr[pl.ds(i * BN, BN), :]
        hnew = jnp.maximum((z2 - mean) * inv * g2_ref[...] + be2_ref[...], 0.0) \
            + h_ref[...]
        hnew_ref[...] = hnew
        xso_ref[...] = xs_ref[...] + hnew


def _layer(h, agg, W1i, b1i, g1i, be1i, W2i, b2i, g2i, be2i, epsp1, xs):
    hmap = lambda p, i: (jnp.where(p == 1, 0, i), 0)
    return pl.pallas_call(
        _layer_body,
        grid=(3, NB),
        in_specs=[
            pl.BlockSpec((BN, H), hmap),
            pl.BlockSpec((2, BN, 128), lambda p, i: (0, jnp.where(p == 0, i, 0), 0)),
            pl.BlockSpec((H, 2 * H), lambda p, i: (0, 0)),
            pl.BlockSpec((1, 2 * H), lambda p, i: (0, 0)),
            pl.BlockSpec((1, 2 * H), lambda p, i: (0, 0)),
            pl.BlockSpec((1, 2 * H), lambda p, i: (0, 0)),
            pl.BlockSpec((2 * H, H), lambda p, i: (0, 0)),
            pl.BlockSpec((1, H), lambda p, i: (0, 0)),
            pl.BlockSpec((1, H), lambda p, i: (0, 0)),
            pl.BlockSpec((1, H), lambda p, i: (0, 0)),
            pl.BlockSpec((1, 1), lambda p, i: (0, 0)),
            pl.BlockSpec((BN, H), lambda p, i: (jnp.where(p == 2, i, 0), 0)),
        ],
        out_specs=[
            pl.BlockSpec((BN, H), lambda p, i: (i, 0)),
            pl.BlockSpec((BN, H), lambda p, i: (i, 0)),
        ],
        out_shape=[
            jax.ShapeDtypeStruct((N, H), jnp.float32),
            jax.ShapeDtypeStruct((N, H), jnp.float32),
        ],
        scratch_shapes=[
            pltpu.VMEM((N, 2 * H), jnp.float32),
            pltpu.VMEM((N, H), jnp.float32),
            pltpu.VMEM((8, 2 * H), jnp.float32),
            pltpu.VMEM((8, H), jnp.float32),
        ],
        compiler_params=pltpu.CompilerParams(
            dimension_semantics=("arbitrary", "arbitrary")),
    )(h, agg, W1i, b1i, g1i, be1i, W2i, b2i, g2i, be2i, epsp1, xs)


def _pool_body(xs_ref, bat_ref, Wf_ref, bf_ref, out_ref, hg_scr):
    i = pl.program_id(0)

    @pl.when(i == 0)
    def _():
        hg_scr[...] = jnp.zeros_like(hg_scr)

    ohB = (lax.broadcasted_iota(jnp.int32, (G, BN), 0)
           == bat_ref[0, 0, :][None, :]).astype(jnp.float32)
    hg_scr[...] += lax.dot_general(ohB, xs_ref[...], (((1,), (0,)), ((), ())),
                                   preferred_element_type=jnp.float32)

    @pl.when(i == NB - 1)
    def _():
        out_ref[...] = lax.dot_general(
            hg_scr[...], Wf_ref[...], (((1,), (0,)), ((), ())),
            preferred_element_type=jnp.float32) + bf_ref[...]


def _pool(xs, bat2, Wf, bf2):
    return pl.pallas_call(
        _pool_body,
        grid=(NB,),
        in_specs=[
            pl.BlockSpec((BN, H), lambda i: (i, 0)),
            pl.BlockSpec((1, 1, BN), lambda i: (i, 0, 0)),
            pl.BlockSpec((H, OUT), lambda i: (0, 0)),
            pl.BlockSpec((1, OUT), lambda i: (0, 0)),
        ],
        out_specs=pl.BlockSpec((G, OUT), lambda i: (0, 0)),
        out_shape=jax.ShapeDtypeStruct((G, OUT), jnp.float32),
        scratch_shapes=[pltpu.VMEM((G, H), jnp.float32)],
        compiler_params=pltpu.CompilerParams(
            dimension_semantics=("arbitrary",)),
    )(xs, bat2, Wf, bf2)


def kernel(x, edge_index, edge_attr, batch, atom_tables, W1, b1, g1, be1,
           W2, b2, eps, g2, be2, Wf, bf):
    del edge_attr
    xfT = (x.astype(jnp.int32).T
           + (jnp.arange(F, dtype=jnp.int32) * VOCAB)[:, None])
    xfT = jnp.concatenate(
        [xfT, jnp.full((16 - F, N), 600, jnp.int32)], axis=0)
    xfT = xfT.reshape(16, NB, BN).swapaxes(0, 1)
    tab = jnp.concatenate(
        [atom_tables.reshape(F * VOCAB, H),
         jnp.zeros((VPAD - F * VOCAB, H), jnp.float32)], axis=0)
    src = edge_index[0].astype(jnp.int32)
    dst = edge_index[1].astype(jnp.int32)
    srcp = jnp.concatenate(
        [src.reshape(NSUB, EP),
         jnp.zeros((NSUB, EPP - EP), jnp.int32)], axis=1)
    dstp = jnp.concatenate(
        [dst.reshape(NSUB, EP),
         jnp.full((NSUB, EPP - EP), N, jnp.int32)], axis=1)
    src2 = jnp.stack([2 * srcp, 2 * srcp + 1]).reshape(2 * NSUB * NBATCH, K)
    dst2 = dstp.reshape(NSUB * NBATCH, K)
    zeros = jnp.zeros((NPAD, 128), jnp.float32)
    bat2 = batch.astype(jnp.int32).reshape(NB, 1, BN)
    bf2 = bf.reshape(1, OUT)

    h = _encoder(xfT, tab)
    xs = h

    for i in range(W1.shape[0]):
        agg = _sc_scatter(h.reshape(2 * N, 128), src2, dst2, zeros)
        agg = agg.reshape(2, NPAD, 128)
        h, xs = _layer(h, agg, W1[i], b1[i].reshape(1, 2 * H),
                       g1[i].reshape(1, 2 * H), be1[i].reshape(1, 2 * H),
                       W2[i], b2[i].reshape(1, H), g2[i].reshape(1, H),
                       be2[i].reshape(1, H),
                       (1.0 + eps[i]).reshape(1, 1), xs)

    return _pool(xs, bat2, Wf, bf2)
